# Initial kernel scaffold; baseline (speedup 1.0000x reference)
import jax
import jax.numpy as jnp
from jax.experimental import pallas as pl


def kernel(x, freqs, emb_weight):
    raise NotImplementedError("write your pallas kernel here")


# speedup vs baseline: 3.6952x; 3.6952x over previous
import jax
import jax.numpy as jnp
from jax.experimental import pallas as pl


_BT = 8


def _add_body(x_ref, emb_ref, o_ref):
    o_ref[...] = x_ref[...] + emb_ref[...]


def kernel(x, freqs, emb_weight):
    del freqs
    nt, nf, d = x.shape
    out = pl.pallas_call(
        _add_body,
        grid=(nt // _BT,),
        in_specs=[
            pl.BlockSpec((_BT, nf, d), lambda i: (i, 0, 0)),
            pl.BlockSpec((nf, d), lambda i: (0, 0)),
        ],
        out_specs=pl.BlockSpec((_BT, nf, d), lambda i: (i, 0, 0)),
        out_shape=jax.ShapeDtypeStruct((nt, nf, d), x.dtype),
    )(x, emb_weight)
    return out
